# baseline (speedup 1.0000x reference)
import functools

import jax
import jax.numpy as jnp
from jax import lax
from jax.experimental import pallas as pl
from jax.experimental.pallas import tpu as pltpu

_K0 = 2547012911
_K1 = 1371500959


def _threefry_bits(cnt):
    k0 = jnp.uint32(_K0)
    k1 = jnp.uint32(_K1)
    ks2 = jnp.uint32((_K0 ^ _K1 ^ 0x1BD11BDA) & 0xFFFFFFFF)
    ks = (k0, k1, ks2)
    x0 = jnp.zeros_like(cnt) + k0
    x1 = cnt + k1
    rot_a = (13, 15, 26, 6)
    rot_b = (17, 29, 16, 24)
    for i, rots in enumerate((rot_a, rot_b, rot_a, rot_b, rot_a)):
        for r in rots:
            x0 = x0 + x1
            x1 = (x1 << jnp.uint32(r)) | (x1 >> jnp.uint32(32 - r))
            x1 = x1 ^ x0
        x0 = x0 + ks[(i + 1) % 3]
        x1 = x1 + ks[(i + 2) % 3] + jnp.uint32(i + 1)
    return x0 ^ x1


def _body(x_ref, o_ref, best_v, best_i, *, ncols, blk):
    pid = pl.program_id(0)
    nprog = pl.num_programs(0)

    @pl.when(pid == 0)
    def _init():
        best_v[...] = jnp.full(best_v.shape, -jnp.inf, best_v.dtype)
        best_i[...] = jnp.zeros(best_i.shape, best_i.dtype)

    x = x_ref[...]
    rows = x.shape[0]
    row = lax.broadcasted_iota(jnp.uint32, (rows, blk), 0)
    col = lax.broadcasted_iota(jnp.uint32, (rows, blk), 1)
    c0 = (pid * blk).astype(jnp.uint32)
    cnt = row * jnp.uint32(ncols) + col + c0
    bits = _threefry_bits(cnt)
    fbits = (bits >> jnp.uint32(9)) | jnp.uint32(0x3F800000)
    u = lax.bitcast_convert_type(fbits, jnp.float32) - 1.0
    g = -jnp.log(-jnp.log(u + 1e-20) + 1e-20)
    s = jnp.log(jnp.maximum(x, 1e-30)) + g
    cidx = lax.broadcasted_iota(jnp.int32, (rows, blk), 1) + pid * blk
    s = jnp.where(cidx < ncols, s, -jnp.inf)
    m = jnp.max(s, axis=1, keepdims=True)
    idx = jnp.min(
        jnp.where(s == m, cidx, jnp.int32(2**31 - 1)), axis=1, keepdims=True
    )
    better = m > best_v[...]
    best_v[...] = jnp.where(better, m, best_v[...])
    best_i[...] = jnp.where(better, idx, best_i[...])

    @pl.when(pid == nprog - 1)
    def _fin():
        o_ref[...] = best_i[...]


@jax.jit
def kernel(x):
    rows, ncols = x.shape
    blk = min(32768, ncols)
    n = pl.cdiv(ncols, blk)
    return pl.pallas_call(
        functools.partial(_body, ncols=ncols, blk=blk),
        grid=(n,),
        in_specs=[pl.BlockSpec((rows, blk), lambda i: (0, i))],
        out_specs=pl.BlockSpec((rows, 1), lambda i: (0, 0)),
        out_shape=jax.ShapeDtypeStruct((rows, 1), jnp.int32),
        scratch_shapes=[
            pltpu.VMEM((rows, 1), jnp.float32),
            pltpu.VMEM((rows, 1), jnp.int32),
        ],
    )(x)


# speedup vs baseline: 1.3859x; 1.3859x over previous
import functools

import jax
import jax.numpy as jnp
from jax import lax
from jax.experimental import pallas as pl
from jax.experimental.pallas import tpu as pltpu

_K0 = 2547012911
_K1 = 1371500959


def _threefry_bits(cnt):
    k0 = jnp.uint32(_K0)
    k1 = jnp.uint32(_K1)
    ks2 = jnp.uint32((_K0 ^ _K1 ^ 0x1BD11BDA) & 0xFFFFFFFF)
    ks = (k0, k1, ks2)
    x0 = jnp.zeros_like(cnt) + k0
    x1 = cnt + k1
    rot_a = (13, 15, 26, 6)
    rot_b = (17, 29, 16, 24)
    for i, rots in enumerate((rot_a, rot_b, rot_a, rot_b, rot_a)):
        for r in rots:
            x0 = x0 + x1
            x1 = (x1 << jnp.uint32(r)) | (x1 >> jnp.uint32(32 - r))
            x1 = x1 ^ x0
        x0 = x0 + ks[(i + 1) % 3]
        x1 = x1 + ks[(i + 2) % 3] + jnp.uint32(i + 1)
    return x0 ^ x1


def _body(x_ref, o_ref, best_v, best_i, *, ncols, blk):
    pid = pl.program_id(0)
    nprog = pl.num_programs(0)

    @pl.when(pid == 0)
    def _init():
        best_v[...] = jnp.full(best_v.shape, -jnp.inf, best_v.dtype)
        best_i[...] = jnp.zeros(best_i.shape, best_i.dtype)

    x = x_ref[...]
    rows = x.shape[0]
    row = lax.broadcasted_iota(jnp.uint32, (rows, blk), 0)
    col = lax.broadcasted_iota(jnp.uint32, (rows, blk), 1)
    c0 = (pid * blk).astype(jnp.uint32)
    cnt = row * jnp.uint32(ncols) + col + c0
    bits = _threefry_bits(cnt)
    fbits = (bits >> jnp.uint32(9)) | jnp.uint32(0x3F800000)
    u = lax.bitcast_convert_type(fbits, jnp.float32) - 1.0
    g = -jnp.log(-jnp.log(u + 1e-20))
    s = jnp.log(jnp.maximum(x, 1e-30)) + g
    cidx = lax.broadcasted_iota(jnp.int32, (rows, blk), 1) + pid * blk
    s = jnp.where(cidx < ncols, s, -jnp.inf)
    m = jnp.max(s, axis=1, keepdims=True)
    idx = jnp.min(
        jnp.where(s == m, cidx, jnp.int32(2**31 - 1)), axis=1, keepdims=True
    )
    better = m > best_v[...]
    best_v[...] = jnp.where(better, m, best_v[...])
    best_i[...] = jnp.where(better, idx, best_i[...])

    @pl.when(pid == nprog - 1)
    def _fin():
        o_ref[...] = best_i[...]


@jax.jit
def kernel(x):
    rows, ncols = x.shape
    blk = min(8192, ncols)
    n = pl.cdiv(ncols, blk)
    return pl.pallas_call(
        functools.partial(_body, ncols=ncols, blk=blk),
        grid=(n,),
        in_specs=[pl.BlockSpec((rows, blk), lambda i: (0, i))],
        out_specs=pl.BlockSpec((rows, 1), lambda i: (0, 0)),
        out_shape=jax.ShapeDtypeStruct((rows, 1), jnp.int32),
        scratch_shapes=[
            pltpu.VMEM((rows, 1), jnp.float32),
            pltpu.VMEM((rows, 1), jnp.int32),
        ],
    )(x)


# speedup vs baseline: 1.6926x; 1.2213x over previous
import functools

import jax
import jax.numpy as jnp
from jax import lax
from jax.experimental import pallas as pl
from jax.experimental.pallas import tpu as pltpu

_K0 = 2547012911
_K1 = 1371500959


def _threefry_bits(cnt):
    k0 = jnp.uint32(_K0)
    k1 = jnp.uint32(_K1)
    ks2 = jnp.uint32((_K0 ^ _K1 ^ 0x1BD11BDA) & 0xFFFFFFFF)
    ks = (k0, k1, ks2)
    x0 = jnp.zeros_like(cnt) + k0
    x1 = cnt + k1
    rot_a = (13, 15, 26, 6)
    rot_b = (17, 29, 16, 24)
    for i, rots in enumerate((rot_a, rot_b, rot_a, rot_b, rot_a)):
        for r in rots:
            x0 = x0 + x1
            x1 = (x1 << jnp.uint32(r)) | (x1 >> jnp.uint32(32 - r))
            x1 = x1 ^ x0
        x0 = x0 + ks[(i + 1) % 3]
        x1 = x1 + ks[(i + 2) % 3] + jnp.uint32(i + 1)
    return x0 ^ x1


def _body(x_ref, o_ref, best_v, best_i, *, ncols, blk):
    pid = pl.program_id(0)
    nprog = pl.num_programs(0)

    @pl.when(pid == 0)
    def _init():
        best_v[...] = jnp.full(best_v.shape, -jnp.inf, best_v.dtype)
        best_i[...] = jnp.zeros(best_i.shape, best_i.dtype)

    x = x_ref[...]
    rows = x.shape[0]
    row = lax.broadcasted_iota(jnp.uint32, (rows, blk), 0)
    col = lax.broadcasted_iota(jnp.uint32, (rows, blk), 1)
    c0 = (pid * blk).astype(jnp.uint32)
    cnt = row * jnp.uint32(ncols) + col + c0
    bits = _threefry_bits(cnt)
    fbits = (bits >> jnp.uint32(9)) | jnp.uint32(0x3F800000)
    u = lax.bitcast_convert_type(fbits, jnp.float32) - 1.0
    g = -jnp.log(-jnp.log(u + 1e-20))
    s = jnp.log(jnp.maximum(x, 1e-30)) + g
    cidx = lax.broadcasted_iota(jnp.int32, (rows, blk), 1) + pid * blk
    s = jnp.where(cidx < ncols, s, -jnp.inf)
    m = jnp.max(s, axis=1, keepdims=True)
    idx = jnp.min(
        jnp.where(s == m, cidx, jnp.int32(2**31 - 1)), axis=1, keepdims=True
    )
    better = m > best_v[...]
    best_v[...] = jnp.where(better, m, best_v[...])
    best_i[...] = jnp.where(better, idx, best_i[...])

    @pl.when(pid == nprog - 1)
    def _fin():
        o_ref[...] = best_i[...]


@jax.jit
def kernel(x):
    rows, ncols = x.shape
    blk = min(4096, ncols)
    n = pl.cdiv(ncols, blk)
    return pl.pallas_call(
        functools.partial(_body, ncols=ncols, blk=blk),
        grid=(n,),
        in_specs=[pl.BlockSpec((rows, blk), lambda i: (0, i))],
        out_specs=pl.BlockSpec((rows, 1), lambda i: (0, 0)),
        out_shape=jax.ShapeDtypeStruct((rows, 1), jnp.int32),
        scratch_shapes=[
            pltpu.VMEM((rows, 1), jnp.float32),
            pltpu.VMEM((rows, 1), jnp.int32),
        ],
    )(x)


# speedup vs baseline: 1.8843x; 1.1132x over previous
import functools

import jax
import jax.numpy as jnp
from jax import lax
from jax.experimental import pallas as pl
from jax.experimental.pallas import tpu as pltpu

_K0 = 2547012911
_K1 = 1371500959


def _threefry_bits(cnt):
    k0 = jnp.uint32(_K0)
    k1 = jnp.uint32(_K1)
    ks2 = jnp.uint32((_K0 ^ _K1 ^ 0x1BD11BDA) & 0xFFFFFFFF)
    ks = (k0, k1, ks2)
    x0 = jnp.zeros_like(cnt) + k0
    x1 = cnt + k1
    rot_a = (13, 15, 26, 6)
    rot_b = (17, 29, 16, 24)
    for i, rots in enumerate((rot_a, rot_b, rot_a, rot_b, rot_a)):
        for r in rots:
            x0 = x0 + x1
            x1 = (x1 << jnp.uint32(r)) | (x1 >> jnp.uint32(32 - r))
            x1 = x1 ^ x0
        x0 = x0 + ks[(i + 1) % 3]
        x1 = x1 + ks[(i + 2) % 3] + jnp.uint32(i + 1)
    return x0 ^ x1


def _body(x_ref, o_ref, acc_ref, idx_ref, *, ncols, blk, ch):
    pid = pl.program_id(0)
    nprog = pl.num_programs(0)
    rows = x_ref.shape[0]
    nch = blk // ch

    @pl.when(pid == 0)
    def _init():
        acc_ref[...] = jnp.full(acc_ref.shape, -jnp.inf, acc_ref.dtype)
        idx_ref[...] = jnp.zeros(idx_ref.shape, idx_ref.dtype)

    row = lax.broadcasted_iota(jnp.uint32, (rows, ch), 0)
    colv = lax.broadcasted_iota(jnp.uint32, (rows, ch), 1)
    base_vec = row * jnp.uint32(ncols) + colv
    cidx0 = lax.broadcasted_iota(jnp.int32, (rows, ch), 1)

    def chunk(j, carry):
        acc, idxa = carry
        c0 = pid * blk + j * ch
        x = x_ref[:, pl.ds(pl.multiple_of(j * ch, ch), ch)]
        cnt = base_vec + c0.astype(jnp.uint32)
        bits = _threefry_bits(cnt)
        fbits = (bits >> jnp.uint32(9)) | jnp.uint32(0x3F800000)
        u = lax.bitcast_convert_type(fbits, jnp.float32) - 1.0
        g = -jnp.log(-jnp.log(u + 1e-20))
        s = jnp.log(jnp.maximum(x, 1e-30)) + g
        cidx = cidx0 + c0
        s = jnp.where(cidx < ncols, s, -jnp.inf)
        better = s > acc
        acc = jnp.where(better, s, acc)
        idxa = jnp.where(better, cidx, idxa)
        return acc, idxa

    carry0 = (acc_ref[...], idx_ref[...])
    acc, idxa = lax.fori_loop(0, nch, chunk, carry0)
    acc_ref[...] = acc
    idx_ref[...] = idxa

    @pl.when(pid == nprog - 1)
    def _fin():
        a = acc_ref[...]
        ia = idx_ref[...]
        m = jnp.max(a, axis=1, keepdims=True)
        o_ref[...] = jnp.min(
            jnp.where(a == m, ia, jnp.int32(2**31 - 1)), axis=1, keepdims=True
        )


@jax.jit
def kernel(x):
    rows, ncols = x.shape
    ch = 128
    blk = min(8192, ncols)
    n = pl.cdiv(ncols, blk)
    return pl.pallas_call(
        functools.partial(_body, ncols=ncols, blk=blk, ch=ch),
        grid=(n,),
        in_specs=[pl.BlockSpec((rows, blk), lambda i: (0, i))],
        out_specs=pl.BlockSpec((rows, 1), lambda i: (0, 0)),
        out_shape=jax.ShapeDtypeStruct((rows, 1), jnp.int32),
        scratch_shapes=[
            pltpu.VMEM((rows, ch), jnp.float32),
            pltpu.VMEM((rows, ch), jnp.int32),
        ],
    )(x)
